# baseline (speedup 1.0000x reference)
import functools

import jax
import jax.numpy as jnp
from jax import lax
from jax.experimental import pallas as pl
from jax.experimental.pallas import tpu as pltpu

_K = 3
_PAD = 1
_EPS = 1e-5
_BB = 16


def _roll_lanes(x, s):
    if s == 0:
        return x
    return jnp.concatenate([x[..., s:], x[..., :s]], axis=-1)


def _make_patch_builder(H, W, Cin):
    HW = H * W

    def build(xb, patch_ref):
        idx = lax.broadcasted_iota(jnp.int32, (1, 1, HW), 2)
        hh = idx // W
        ww = idx % W
        for ky in range(_K):
            for kx in range(_K):
                dy = ky - _PAD
                dx = kx - _PAD
                t = ky * _K + kx
                xs = _roll_lanes(xb, dy * W + dx)
                conds = []
                if dy < 0:
                    conds.append(hh >= -dy)
                if dy > 0:
                    conds.append(hh < H - dy)
                if dx < 0:
                    conds.append(ww >= -dx)
                if dx > 0:
                    conds.append(ww < W - dx)
                if conds:
                    m = conds[0]
                    for c in conds[1:]:
                        m = jnp.logical_and(m, c)
                    xs = jnp.where(m, xs, jnp.bfloat16(0))
                patch_ref[:, t * Cin:(t + 1) * Cin, :] = xs

    return build


@jax.jit
def _conv_bn(x_nchw, weight, gamma, beta):
    N, Cin, H, W = x_nchw.shape
    Cout = weight.shape[0]
    HW = H * W
    KKC = _K * _K * Cin
    B = _BB if N % _BB == 0 else 1
    G = N // B

    x2 = x_nchw.reshape(N, Cin, HW)
    w2 = jnp.transpose(weight, (0, 2, 3, 1)).reshape(Cout, KKC).astype(jnp.bfloat16)
    g_row = gamma.reshape(1, Cout)
    b_row = beta.reshape(1, Cout)

    build_patch = _make_patch_builder(H, W, Cin)

    cparams = pltpu.CompilerParams(
        dimension_semantics=("arbitrary",),
        vmem_limit_bytes=100 * 1024 * 1024,
    )

    BS = 32 if N % 32 == 0 else B
    GS = N // BS
    SUB = 8 if BS % 8 == 0 else BS

    def stats_body(x_ref, w_ref, conv_ref, stats_ref, patch_ref):
        s1 = jnp.zeros((Cout, 1), jnp.float32)
        s2 = jnp.zeros((Cout, 1), jnp.float32)
        for c in range(BS // SUB):
            xb = x_ref[c * SUB:(c + 1) * SUB].astype(jnp.bfloat16)
            build_patch(xb, patch_ref)
            for b in range(SUB):
                acc = jnp.dot(w_ref[...], patch_ref[b],
                              preferred_element_type=jnp.float32)
                s1 = s1 + jnp.sum(acc, axis=1, keepdims=True)
                s2 = s2 + jnp.sum(acc * acc, axis=1, keepdims=True)
                conv_ref[c * SUB + b] = acc.astype(jnp.bfloat16)
        stats_ref[0] = jnp.concatenate([s1, s2], axis=1)

    conv_bf, stats = pl.pallas_call(
        stats_body,
        grid=(GS,),
        in_specs=[
            pl.BlockSpec((BS, Cin, HW), lambda i: (i, 0, 0)),
            pl.BlockSpec((Cout, KKC), lambda i: (0, 0)),
        ],
        out_specs=[
            pl.BlockSpec((BS, Cout, HW), lambda i: (i, 0, 0)),
            pl.BlockSpec((1, Cout, 2), lambda i: (i, 0, 0)),
        ],
        out_shape=[
            jax.ShapeDtypeStruct((N, Cout, HW), jnp.bfloat16),
            jax.ShapeDtypeStruct((GS, Cout, 2), jnp.float32),
        ],
        scratch_shapes=[pltpu.VMEM((SUB, KKC, HW), jnp.bfloat16)],
        compiler_params=cparams,
    )(x2, w2)

    inv_cnt = 1.0 / float(N * HW)

    def apply_body(conv_ref, stats_ref, g_ref, b_ref, o_ref):
        tot = jnp.sum(stats_ref[...], axis=0)
        mean = tot[:, 0:1] * inv_cnt
        var = tot[:, 1:2] * inv_cnt - mean * mean
        inv = lax.rsqrt(var + _EPS)
        gcol = jnp.transpose(g_ref[...])
        bcol = jnp.transpose(b_ref[...])
        sc = gcol * inv
        sh = bcol - mean * sc
        o_ref[...] = conv_ref[...].astype(jnp.float32) * sc + sh

    y = pl.pallas_call(
        apply_body,
        grid=(G,),
        in_specs=[
            pl.BlockSpec((B, Cout, HW), lambda i: (i, 0, 0)),
            pl.BlockSpec((GS, Cout, 2), lambda i: (0, 0, 0)),
            pl.BlockSpec((1, Cout), lambda i: (0, 0)),
            pl.BlockSpec((1, Cout), lambda i: (0, 0)),
        ],
        out_specs=pl.BlockSpec((B, Cout, HW), lambda i: (i, 0, 0)),
        out_shape=jax.ShapeDtypeStruct((N, Cout, HW), jnp.float32),
        compiler_params=cparams,
    )(conv_bf, stats, g_row, b_row)

    return y.reshape(N, Cout, H, W)


def kernel(x_nchw, weight, bias, gamma, beta):
    del bias
    return _conv_bn(x_nchw, weight, gamma, beta)


# speedup vs baseline: 1.0899x; 1.0899x over previous
import functools

import jax
import jax.numpy as jnp
from jax import lax
from jax.experimental import pallas as pl
from jax.experimental.pallas import tpu as pltpu

_K = 3
_PAD = 1
_EPS = 1e-5
_BB = 16


def _roll_lanes(x, s):
    if s == 0:
        return x
    return jnp.concatenate([x[..., s:], x[..., :s]], axis=-1)


def _make_patch_builder(H, W, Cin):
    HW = H * W

    def build(xb, patch_ref):
        idx = lax.broadcasted_iota(jnp.int32, (1, 1, HW), 2)
        hh = idx // W
        ww = idx % W
        for ky in range(_K):
            for kx in range(_K):
                dy = ky - _PAD
                dx = kx - _PAD
                t = ky * _K + kx
                xs = _roll_lanes(xb, dy * W + dx)
                conds = []
                if dy < 0:
                    conds.append(hh >= -dy)
                if dy > 0:
                    conds.append(hh < H - dy)
                if dx < 0:
                    conds.append(ww >= -dx)
                if dx > 0:
                    conds.append(ww < W - dx)
                if conds:
                    m = conds[0]
                    for c in conds[1:]:
                        m = jnp.logical_and(m, c)
                    xs = jnp.where(m, xs, jnp.bfloat16(0))
                patch_ref[:, t * Cin:(t + 1) * Cin, :] = xs

    return build


@jax.jit
def _conv_bn(x_nchw, weight, gamma, beta):
    N, Cin, H, W = x_nchw.shape
    Cout = weight.shape[0]
    HW = H * W
    KKC = _K * _K * Cin
    B = _BB if N % _BB == 0 else 1
    G = N // B

    x2 = x_nchw.reshape(N, Cin, HW)
    w2 = jnp.transpose(weight, (0, 2, 3, 1)).reshape(Cout, KKC).astype(jnp.bfloat16)
    g_row = gamma.reshape(1, Cout)
    b_row = beta.reshape(1, Cout)

    build_patch = _make_patch_builder(H, W, Cin)

    cparams = pltpu.CompilerParams(
        dimension_semantics=("arbitrary",),
        vmem_limit_bytes=100 * 1024 * 1024,
    )

    GS = G

    def stats_body(x_ref, w_ref, conv_ref, stats_ref, patch_ref):
        s1 = jnp.zeros((Cout, 1), jnp.float32)
        s2 = jnp.zeros((Cout, 1), jnp.float32)
        xb = x_ref[...].astype(jnp.bfloat16)
        build_patch(xb, patch_ref)
        for b in range(B):
            acc = jnp.dot(w_ref[...], patch_ref[b],
                          preferred_element_type=jnp.float32)
            s1 = s1 + jnp.sum(acc, axis=1, keepdims=True)
            s2 = s2 + jnp.sum(acc * acc, axis=1, keepdims=True)
            conv_ref[b] = acc.astype(jnp.bfloat16)
        stats_ref[0] = jnp.concatenate([s1, s2], axis=1)

    conv_bf, stats = pl.pallas_call(
        stats_body,
        grid=(G,),
        in_specs=[
            pl.BlockSpec((B, Cin, HW), lambda i: (i, 0, 0)),
            pl.BlockSpec((Cout, KKC), lambda i: (0, 0)),
        ],
        out_specs=[
            pl.BlockSpec((B, Cout, HW), lambda i: (i, 0, 0)),
            pl.BlockSpec((1, Cout, 2), lambda i: (i, 0, 0)),
        ],
        out_shape=[
            jax.ShapeDtypeStruct((N, Cout, HW), jnp.bfloat16),
            jax.ShapeDtypeStruct((G, Cout, 2), jnp.float32),
        ],
        scratch_shapes=[pltpu.VMEM((B, KKC, HW), jnp.bfloat16)],
        compiler_params=cparams,
    )(x2, w2)

    inv_cnt = 1.0 / float(N * HW)

    def apply_body(conv_ref, stats_ref, g_ref, b_ref, o_ref):
        tot = jnp.sum(stats_ref[...], axis=0)
        mean = tot[:, 0:1] * inv_cnt
        var = tot[:, 1:2] * inv_cnt - mean * mean
        inv = lax.rsqrt(var + _EPS)
        gcol = jnp.transpose(g_ref[...])
        bcol = jnp.transpose(b_ref[...])
        sc = gcol * inv
        sh = bcol - mean * sc
        o_ref[...] = conv_ref[...].astype(jnp.float32) * sc + sh

    y = pl.pallas_call(
        apply_body,
        grid=(G,),
        in_specs=[
            pl.BlockSpec((B, Cout, HW), lambda i: (i, 0, 0)),
            pl.BlockSpec((GS, Cout, 2), lambda i: (0, 0, 0)),
            pl.BlockSpec((1, Cout), lambda i: (0, 0)),
            pl.BlockSpec((1, Cout), lambda i: (0, 0)),
        ],
        out_specs=pl.BlockSpec((B, Cout, HW), lambda i: (i, 0, 0)),
        out_shape=jax.ShapeDtypeStruct((N, Cout, HW), jnp.float32),
        compiler_params=cparams,
    )(conv_bf, stats, g_row, b_row)

    return y.reshape(N, Cout, H, W)


def kernel(x_nchw, weight, bias, gamma, beta):
    del bias
    return _conv_bn(x_nchw, weight, gamma, beta)
